# baseline (speedup 1.0000x reference)
import functools

import jax
import jax.numpy as jnp
from jax import lax
from jax.experimental import pallas as pl
from jax.experimental.pallas import tpu as pltpu
from jax.experimental.pallas import tpu_sc as plsc

N = 10000
D = 128
E = 320000

NC = 2
NS = 16
NW = NC * NS
G = E // 128
GT = G // NW
GX = G - GT * NW
CH = 80
ROWS = 624
ROWS_LAST = N - (NS - 1) * ROWS


def _copy_rows(src, dst, s):
    lo = s * ROWS

    @pl.when(s < NS - 1)
    def _():
        pltpu.sync_copy(src.at[pl.ds(lo, ROWS)], dst.at[pl.ds(lo, ROWS)])

    @pl.when(s == NS - 1)
    def _():
        base = (NS - 1) * ROWS
        pltpu.sync_copy(src.at[pl.ds(base, ROWS_LAST)],
                        dst.at[pl.ds(base, ROWS_LAST)])

_mesh = plsc.VectorSubcoreMesh(core_axis_name="c", subcore_axis_name="s")


@functools.partial(
    pl.kernel,
    out_type=(
        jax.ShapeDtypeStruct((N, 16), jnp.float32),
        jax.ShapeDtypeStruct((N, 16), jnp.float32),
    ),
    mesh=_mesh,
    scratch_types=[
        pltpu.VMEM((GT + 1, 2, 128), jnp.int32),
        pltpu.VMEM((128, 16), jnp.float32),
        pltpu.VMEM_SHARED((N, 16), jnp.float32),
        pltpu.SemaphoreType.DMA,
    ],
    compiler_params=pltpu.CompilerParams(use_tc_tiling_on_sc=False),
)
def _deg_kernel(e3_hbm, ones_hbm, zeros_hbm, d0_hbm, d1_hbm,
                e3_v, ones_v, acc, sem):
    c = lax.axis_index("c")
    s = lax.axis_index("s")
    t = c * NS + s
    g0 = t * GT + jnp.minimum(t, GX)
    d1 = pltpu.async_copy(ones_hbm, ones_v, sem)
    d2 = pltpu.async_copy(e3_hbm.at[pl.ds(g0, GT)], e3_v.at[pl.ds(0, GT)], sem)
    _copy_rows(zeros_hbm, acc, s)
    d1.wait()
    d2.wait()

    @pl.when(t < GX)
    def _():
        pltpu.sync_copy(e3_hbm.at[pl.ds(g0 + GT, 1)],
                        e3_v.at[pl.ds(GT, 1)])

    plsc.subcore_barrier()

    @pl.loop(0, GT)
    def _(g):
        pltpu.async_copy(ones_v, acc.at[e3_v.at[g, 1]], sem, add=True)

    @pl.when(t < GX)
    def _():
        pltpu.async_copy(ones_v, acc.at[e3_v.at[GT, 1]], sem, add=True)

    @pl.loop(0, GT)
    def _(g):
        pltpu.make_async_copy(ones_v, acc.at[e3_v.at[0, 1]], sem).wait()

    @pl.when(t < GX)
    def _():
        pltpu.make_async_copy(ones_v, acc.at[e3_v.at[0, 1]], sem).wait()

    plsc.subcore_barrier()

    @pl.when(c == 0)
    def _():
        _copy_rows(acc, d0_hbm, s)

    @pl.when(c == 1)
    def _():
        _copy_rows(acc, d1_hbm, s)


@functools.partial(
    pl.kernel,
    out_type=(
        jax.ShapeDtypeStruct((N, D), jnp.float32),
        jax.ShapeDtypeStruct((N, D), jnp.float32),
    ),
    mesh=_mesh,
    scratch_types=[
        pltpu.VMEM((4, 2, 128), jnp.int32),
        pltpu.VMEM((2, 128, D), jnp.float32),
        pltpu.VMEM_SHARED((N, D), jnp.float32),
        pltpu.SemaphoreType.DMA,
        pltpu.SemaphoreType.DMA,
    ],
    compiler_params=pltpu.CompilerParams(use_tc_tiling_on_sc=False),
)
def _agg_kernel(hp_hbm, e3_hbm, s0_hbm, s1_hbm,
                idx_v, rows_v, acc, gsem, isem):
    c = lax.axis_index("c")
    s = lax.axis_index("s")
    t = c * NS + s
    g0 = t * GT + jnp.minimum(t, GX)
    gg = GT + (t < GX).astype(jnp.int32)

    for m in range(3):
        pltpu.async_copy(e3_hbm.at[pl.ds(g0 + m, 1)], idx_v.at[pl.ds(m, 1)],
                         isem)

    @pl.when(c == 0)
    def _():
        _copy_rows(hp_hbm, acc, s)

    @pl.when(c == 1)
    def _():
        z = jnp.zeros((16,), jnp.float32)

        @pl.loop(0, 128)
        def _(r):
            for l in range(D // 16):
                rows_v[0, r, pl.ds(16 * l, 16)] = z

        lo = s * ROWS
        for k in range(ROWS // 128):
            pltpu.sync_copy(rows_v.at[0], acc.at[pl.ds(lo + k * 128, 128)])
        rem = ROWS % 128
        if rem:
            pltpu.sync_copy(rows_v.at[0].at[pl.ds(0, rem)],
                            acc.at[pl.ds(lo + (ROWS // 128) * 128, rem)])

        @pl.when(s == NS - 1)
        def _():
            base = (NS - 1) * ROWS + ROWS
            extra = ROWS_LAST - ROWS
            pltpu.sync_copy(rows_v.at[0].at[pl.ds(0, extra)],
                            acc.at[pl.ds(base, extra)])

    pltpu.make_async_copy(e3_hbm.at[pl.ds(g0, 1)], idx_v.at[pl.ds(0, 1)],
                          isem).wait()
    pltpu.async_copy(hp_hbm.at[idx_v.at[0, 0]], rows_v.at[0], gsem)
    plsc.subcore_barrier()

    @pl.loop(0, gg)
    def _(g):
        m = g % 4
        b = g % 2

        @pl.when(g + 1 < gg)
        def _():
            mn = (g + 1) % 4
            pltpu.make_async_copy(e3_hbm.at[pl.ds(g0 + g + 1, 1)],
                                  idx_v.at[pl.ds(mn, 1)], isem).wait()
            pltpu.async_copy(hp_hbm.at[idx_v.at[mn, 0]], rows_v.at[1 - b],
                             gsem)

        @pl.when(g + 3 < gg)
        def _():
            mf = (g + 3) % 4
            pltpu.async_copy(e3_hbm.at[pl.ds(g0 + g + 3, 1)],
                             idx_v.at[pl.ds(mf, 1)], isem)

        pltpu.make_async_copy(hp_hbm.at[idx_v.at[m, 0]], rows_v.at[b],
                              gsem).wait()
        pltpu.sync_copy(rows_v.at[b], acc.at[idx_v.at[m, 1]], add=True)

    plsc.subcore_barrier()

    @pl.when(c == 0)
    def _():
        _copy_rows(acc, s0_hbm, s)

    @pl.when(c == 1)
    def _():
        _copy_rows(acc, s1_hbm, s)


_BM = 1000


def _mm_body(x_ref, w_ref, h_ref):
    h_ref[...] = jnp.dot(x_ref[...], w_ref[...],
                         preferred_element_type=jnp.float32)


def _scale_body(h_ref, d0_ref, d1_ref, hp_ref, dinv_ref):
    deg = d0_ref[...] + d1_ref[...] + 1.0
    dinv8 = lax.rsqrt(deg)
    hp_ref[...] = h_ref[...] * dinv8[:, 0:1]
    dinv_ref[...] = dinv8


def _combine_body(s0_ref, s1_ref, dinv_ref, b_ref, o_ref):
    tot = s0_ref[...] + s1_ref[...]
    o_ref[...] = jnp.maximum(tot * dinv_ref[:, 0:1] + b_ref[...], 0.0)


def kernel(x, edge_index, W, b):
    e3 = edge_index.reshape(2, G, 128).transpose(1, 0, 2)
    ones8 = jnp.ones((128, 16), jnp.float32)
    zeros8 = jnp.zeros((N, 16), jnp.float32)

    d0, d1 = _deg_kernel(e3, ones8, zeros8)

    h = pl.pallas_call(
        _mm_body,
        grid=(N // _BM,),
        in_specs=[
            pl.BlockSpec((_BM, D), lambda i: (i, 0)),
            pl.BlockSpec((D, D), lambda i: (0, 0)),
        ],
        out_specs=pl.BlockSpec((_BM, D), lambda i: (i, 0)),
        out_shape=jax.ShapeDtypeStruct((N, D), jnp.float32),
    )(x, W)

    hp, dinv8 = pl.pallas_call(
        _scale_body,
        grid=(N // _BM,),
        in_specs=[
            pl.BlockSpec((_BM, D), lambda i: (i, 0)),
            pl.BlockSpec((_BM, 16), lambda i: (i, 0)),
            pl.BlockSpec((_BM, 16), lambda i: (i, 0)),
        ],
        out_specs=[
            pl.BlockSpec((_BM, D), lambda i: (i, 0)),
            pl.BlockSpec((_BM, 16), lambda i: (i, 0)),
        ],
        out_shape=[
            jax.ShapeDtypeStruct((N, D), jnp.float32),
            jax.ShapeDtypeStruct((N, 16), jnp.float32),
        ],
    )(h, d0, d1)

    s0, s1 = _agg_kernel(hp, e3)

    out = pl.pallas_call(
        _combine_body,
        grid=(N // _BM,),
        in_specs=[
            pl.BlockSpec((_BM, D), lambda i: (i, 0)),
            pl.BlockSpec((_BM, D), lambda i: (i, 0)),
            pl.BlockSpec((_BM, 16), lambda i: (i, 0)),
            pl.BlockSpec((D,), lambda i: (0,)),
        ],
        out_specs=pl.BlockSpec((_BM, D), lambda i: (i, 0)),
        out_shape=jax.ShapeDtypeStruct((N, D), jnp.float32),
    )(s0, s1, dinv8, b)
    return out


# speedup vs baseline: 1.0026x; 1.0026x over previous
import functools

import jax
import jax.numpy as jnp
from jax import lax
from jax.experimental import pallas as pl
from jax.experimental.pallas import tpu as pltpu
from jax.experimental.pallas import tpu_sc as plsc

N = 10000
D = 128
E = 320000

NC = 2
NS = 16
NW = NC * NS
G = E // 128
GT = G // NW
GX = G - GT * NW
CH = 80
ROWS = 624
ROWS_LAST = N - (NS - 1) * ROWS


def _copy_rows(src, dst, s):
    lo = s * ROWS

    @pl.when(s < NS - 1)
    def _():
        pltpu.sync_copy(src.at[pl.ds(lo, ROWS)], dst.at[pl.ds(lo, ROWS)])

    @pl.when(s == NS - 1)
    def _():
        base = (NS - 1) * ROWS
        pltpu.sync_copy(src.at[pl.ds(base, ROWS_LAST)],
                        dst.at[pl.ds(base, ROWS_LAST)])

_mesh = plsc.VectorSubcoreMesh(core_axis_name="c", subcore_axis_name="s")


@functools.partial(
    pl.kernel,
    out_type=(
        jax.ShapeDtypeStruct((N, 16), jnp.float32),
        jax.ShapeDtypeStruct((N, 16), jnp.float32),
    ),
    mesh=_mesh,
    scratch_types=[
        pltpu.VMEM((GT + 1, 2, 128), jnp.int32),
        pltpu.VMEM((128, 16), jnp.float32),
        pltpu.VMEM_SHARED((N, 16), jnp.float32),
        pltpu.SemaphoreType.DMA,
    ],
    compiler_params=pltpu.CompilerParams(use_tc_tiling_on_sc=False),
)
def _deg_kernel(e3_hbm, ones_hbm, zeros_hbm, d0_hbm, d1_hbm,
                e3_v, ones_v, acc, sem):
    c = lax.axis_index("c")
    s = lax.axis_index("s")
    t = c * NS + s
    g0 = t * GT + jnp.minimum(t, GX)
    d1 = pltpu.async_copy(ones_hbm, ones_v, sem)
    d2 = pltpu.async_copy(e3_hbm.at[pl.ds(g0, GT)], e3_v.at[pl.ds(0, GT)], sem)
    _copy_rows(zeros_hbm, acc, s)
    d1.wait()
    d2.wait()

    @pl.when(t < GX)
    def _():
        pltpu.sync_copy(e3_hbm.at[pl.ds(g0 + GT, 1)],
                        e3_v.at[pl.ds(GT, 1)])

    plsc.subcore_barrier()

    @pl.loop(0, GT)
    def _(g):
        pltpu.async_copy(ones_v, acc.at[e3_v.at[g, 1]], sem, add=True)

    @pl.when(t < GX)
    def _():
        pltpu.async_copy(ones_v, acc.at[e3_v.at[GT, 1]], sem, add=True)

    @pl.loop(0, GT)
    def _(g):
        pltpu.make_async_copy(ones_v, acc.at[e3_v.at[0, 1]], sem).wait()

    @pl.when(t < GX)
    def _():
        pltpu.make_async_copy(ones_v, acc.at[e3_v.at[0, 1]], sem).wait()

    plsc.subcore_barrier()

    @pl.when(c == 0)
    def _():
        _copy_rows(acc, d0_hbm, s)

    @pl.when(c == 1)
    def _():
        _copy_rows(acc, d1_hbm, s)


@functools.partial(
    pl.kernel,
    out_type=(
        jax.ShapeDtypeStruct((N, D), jnp.float32),
        jax.ShapeDtypeStruct((N, D), jnp.float32),
    ),
    mesh=_mesh,
    scratch_types=[
        pltpu.VMEM((4, 2, 128), jnp.int32),
        pltpu.VMEM((2, 128, D), jnp.float32),
        pltpu.VMEM_SHARED((N, D), jnp.float32),
        pltpu.SemaphoreType.DMA,
        pltpu.SemaphoreType.DMA,
    ],
    compiler_params=pltpu.CompilerParams(use_tc_tiling_on_sc=False),
)
def _agg_kernel(hp_hbm, e3_hbm, s0_hbm, s1_hbm,
                idx_v, rows_v, acc, gsem, isem):
    c = lax.axis_index("c")
    s = lax.axis_index("s")
    t = c * NS + s
    g0 = t * GT + jnp.minimum(t, GX)
    gg = GT + (t < GX).astype(jnp.int32)

    for m in range(3):
        pltpu.async_copy(e3_hbm.at[pl.ds(g0 + m, 1)], idx_v.at[pl.ds(m, 1)],
                         isem)

    @pl.when(c == 0)
    def _():
        _copy_rows(hp_hbm, acc, s)

    @pl.when(c == 1)
    def _():
        z = jnp.zeros((16,), jnp.float32)

        @pl.loop(0, 128)
        def _(r):
            for l in range(D // 16):
                rows_v[0, r, pl.ds(16 * l, 16)] = z

        lo = s * ROWS
        for k in range(ROWS // 128):
            pltpu.sync_copy(rows_v.at[0], acc.at[pl.ds(lo + k * 128, 128)])
        rem = ROWS % 128
        if rem:
            pltpu.sync_copy(rows_v.at[0].at[pl.ds(0, rem)],
                            acc.at[pl.ds(lo + (ROWS // 128) * 128, rem)])

        @pl.when(s == NS - 1)
        def _():
            base = (NS - 1) * ROWS + ROWS
            extra = ROWS_LAST - ROWS
            pltpu.sync_copy(rows_v.at[0].at[pl.ds(0, extra)],
                            acc.at[pl.ds(base, extra)])

    pltpu.make_async_copy(e3_hbm.at[pl.ds(g0, 1)], idx_v.at[pl.ds(0, 1)],
                          isem).wait()
    pltpu.async_copy(hp_hbm.at[idx_v.at[0, 0]], rows_v.at[0], gsem)
    plsc.subcore_barrier()

    @pl.loop(0, gg)
    def _(g):
        m = g % 4
        b = g % 2

        @pl.when(g + 1 < gg)
        def _():
            mn = (g + 1) % 4
            pltpu.make_async_copy(e3_hbm.at[pl.ds(g0 + g + 1, 1)],
                                  idx_v.at[pl.ds(mn, 1)], isem).wait()
            pltpu.async_copy(hp_hbm.at[idx_v.at[mn, 0]], rows_v.at[1 - b],
                             gsem)

        @pl.when(g + 3 < gg)
        def _():
            mf = (g + 3) % 4
            pltpu.async_copy(e3_hbm.at[pl.ds(g0 + g + 3, 1)],
                             idx_v.at[pl.ds(mf, 1)], isem)

        pltpu.make_async_copy(hp_hbm.at[idx_v.at[m, 0]], rows_v.at[b],
                              gsem).wait()
        pltpu.sync_copy(rows_v.at[b], acc.at[idx_v.at[m, 1]], add=True)

    plsc.subcore_barrier()

    @pl.when(c == 0)
    def _():
        _copy_rows(acc, s0_hbm, s)

    @pl.when(c == 1)
    def _():
        _copy_rows(acc, s1_hbm, s)


_BM = 1000


def _mm_body(x_ref, w_ref, d0_ref, d1_ref, hp_ref, dinv_ref):
    deg = d0_ref[...] + d1_ref[...] + 1.0
    dinv8 = lax.rsqrt(deg)
    h = jnp.dot(x_ref[...], w_ref[...], preferred_element_type=jnp.float32)
    hp_ref[...] = h * dinv8[:, 0:1]
    dinv_ref[...] = dinv8


def _combine_body(s0_ref, s1_ref, dinv_ref, b_ref, o_ref):
    tot = s0_ref[...] + s1_ref[...]
    o_ref[...] = jnp.maximum(tot * dinv_ref[:, 0:1] + b_ref[...], 0.0)


def kernel(x, edge_index, W, b):
    e3 = edge_index.reshape(2, G, 128).transpose(1, 0, 2)
    ones8 = jnp.ones((128, 16), jnp.float32)
    zeros8 = jnp.zeros((N, 16), jnp.float32)

    d0, d1 = _deg_kernel(e3, ones8, zeros8)

    hp, dinv8 = pl.pallas_call(
        _mm_body,
        grid=(N // _BM,),
        in_specs=[
            pl.BlockSpec((_BM, D), lambda i: (i, 0)),
            pl.BlockSpec((D, D), lambda i: (0, 0)),
            pl.BlockSpec((_BM, 16), lambda i: (i, 0)),
            pl.BlockSpec((_BM, 16), lambda i: (i, 0)),
        ],
        out_specs=[
            pl.BlockSpec((_BM, D), lambda i: (i, 0)),
            pl.BlockSpec((_BM, 16), lambda i: (i, 0)),
        ],
        out_shape=[
            jax.ShapeDtypeStruct((N, D), jnp.float32),
            jax.ShapeDtypeStruct((N, 16), jnp.float32),
        ],
    )(x, W, d0, d1)

    s0, s1 = _agg_kernel(hp, e3)

    out = pl.pallas_call(
        _combine_body,
        grid=(N // _BM,),
        in_specs=[
            pl.BlockSpec((_BM, D), lambda i: (i, 0)),
            pl.BlockSpec((_BM, D), lambda i: (i, 0)),
            pl.BlockSpec((_BM, 16), lambda i: (i, 0)),
            pl.BlockSpec((D,), lambda i: (0,)),
        ],
        out_specs=pl.BlockSpec((_BM, D), lambda i: (i, 0)),
        out_shape=jax.ShapeDtypeStruct((N, D), jnp.float32),
    )(s0, s1, dinv8, b)
    return out
